# baseline (speedup 1.0000x reference)
import functools
import jax
import jax.numpy as jnp
from jax import lax
from jax.experimental import pallas as pl
from jax.experimental.pallas import tpu as pltpu
from jax.experimental.pallas import tpu_sc as plsc

_N = 10000
_E = 320000
_D = 128
_NC = 2
_NS = 16
_NW = _NC * _NS
_CH = 200
_NCHUNK = 50
_EPT = _NCHUNK * _CH
_EPAD = _NW * _EPT
_NPAD = 10240
_RPT = _NPAD // _NS


def _seg_sum_body(y_hbm, src_hbm, dst_hbm, zer_hbm, out_hbm,
                  acc, srcb0, dstb0, srcb1, dstb1, gbuf, gsem, isem0, isem1):
    c = lax.axis_index("c")
    s = lax.axis_index("s")
    pltpu.sync_copy(zer_hbm, acc.at[pl.ds(s * _RPT, _RPT)])
    plsc.subcore_barrier()

    cbase = (c * _NS + s) * _EPT

    def istart(g, sb, db, isem):
        off = cbase + g * _CH
        pltpu.async_copy(src_hbm.at[pl.ds(off, _CH)], sb, isem)
        pltpu.async_copy(dst_hbm.at[pl.ds(off, _CH)], db, isem)

    def iwait(g, sb, db, isem):
        off = cbase + g * _CH
        pltpu.make_async_copy(src_hbm.at[pl.ds(off, _CH)], sb, isem).wait()
        pltpu.make_async_copy(dst_hbm.at[pl.ds(off, _CH)], db, isem).wait()

    istart(0, srcb0, dstb0, isem0)
    istart(1, srcb1, dstb1, isem1)

    def chunk(g, sb, db, isem, k):
        iwait(g, sb, db, isem)
        pltpu.async_copy(y_hbm.at[sb], gbuf, gsem).wait()
        pltpu.sync_copy(gbuf, acc.at[db], add=True)

        @pl.when(k < _NCHUNK // 2 - 1)
        def _():
            istart(g + 2, sb, db, isem)

    def body(k, carry):
        chunk(2 * k, srcb0, dstb0, isem0, k)
        chunk(2 * k + 1, srcb1, dstb1, isem1, k)
        return carry

    lax.fori_loop(0, _NCHUNK // 2, body, 0)
    plsc.subcore_barrier()
    pltpu.sync_copy(acc.at[pl.ds(s * _RPT, _RPT)],
                    out_hbm.at[c, pl.ds(s * _RPT, _RPT)])


_seg_sum = functools.partial(
    pl.kernel,
    out_type=jax.ShapeDtypeStruct((_NC, _NPAD, _D), jnp.float32),
    mesh=plsc.VectorSubcoreMesh(core_axis_name="c", subcore_axis_name="s"),
    scratch_types=[
        pltpu.VMEM_SHARED((_NPAD, _D), jnp.float32),
        pltpu.VMEM((_CH,), jnp.int32),
        pltpu.VMEM((_CH,), jnp.int32),
        pltpu.VMEM((_CH,), jnp.int32),
        pltpu.VMEM((_CH,), jnp.int32),
        pltpu.VMEM((_CH, _D), jnp.float32),
        pltpu.SemaphoreType.DMA,
        pltpu.SemaphoreType.DMA,
        pltpu.SemaphoreType.DMA,
    ],
)(_seg_sum_body)


_BLK = 1000


def _root_body(x_ref, wt_ref, br_ref, o_ref):
    o_ref[...] = (jnp.dot(x_ref[...], wt_ref[...],
                          preferred_element_type=jnp.float32) + br_ref[...])


def _root(x, w_root, b_rel):
    return pl.pallas_call(
        _root_body,
        grid=(_N // _BLK,),
        in_specs=[
            pl.BlockSpec((_BLK, _D), lambda i: (i, 0)),
            pl.BlockSpec((_D, _D), lambda i: (0, 0)),
            pl.BlockSpec((1, _D), lambda i: (0, 0)),
        ],
        out_specs=pl.BlockSpec((_BLK, _D), lambda i: (i, 0)),
        out_shape=jax.ShapeDtypeStruct((_N, _D), jnp.float32),
    )(x, w_root, b_rel.reshape(1, _D))


def _comb_body(final, p_ref, r_ref, wr_ref, g_ref, be_ref,
               wl_ref, bl_ref, o_ref):
    h = (jnp.dot(p_ref[0] + p_ref[1], wr_ref[...],
                 preferred_element_type=jnp.float32) + r_ref[...])
    m = jnp.mean(h, axis=-1, keepdims=True)
    v = jnp.mean((h - m) * (h - m), axis=-1, keepdims=True)
    ln = (h - m) * lax.rsqrt(v + 1e-5) * g_ref[...] + be_ref[...]
    if final:
        o_ref[...] = (jnp.dot(ln, wl_ref[...],
                              preferred_element_type=jnp.float32)
                      + bl_ref[...])
    else:
        o_ref[...] = ln


def _comb(p, r, w_rel, g, be, w_lin, b_lin, final):
    vec = pl.BlockSpec((1, _D), lambda i: (0, 0))
    mat = pl.BlockSpec((_D, _D), lambda i: (0, 0))
    return pl.pallas_call(
        functools.partial(_comb_body, final),
        grid=(_N // _BLK,),
        in_specs=[
            pl.BlockSpec((2, _BLK, _D), lambda i: (0, i, 0)),
            pl.BlockSpec((_BLK, _D), lambda i: (i, 0)),
            mat, vec, vec, mat, vec,
        ],
        out_specs=pl.BlockSpec((_BLK, _D), lambda i: (i, 0)),
        out_shape=jax.ShapeDtypeStruct((_N, _D), jnp.float32),
    )(p, r, w_rel, g.reshape(1, _D), be.reshape(1, _D), w_lin,
      b_lin.reshape(1, _D))


def kernel(x, edge_index, batch,
           W1_rel, b1_rel, W1_root, g1, be1,
           W2_rel, b2_rel, W2_root, g2, be2,
           W3_rel, b3_rel, W3_root, g3, be3,
           Wlin, blin):
    del batch
    src = edge_index[0]
    dst = edge_index[1]
    zer = jnp.zeros((_RPT, _D), jnp.float32)

    r = _root(x, W1_root, b1_rel)
    p = _seg_sum(x, src, dst, zer)
    h = _comb(p, r, W1_rel, g1, be1, Wlin, blin, False)
    r = _root(h, W2_root, b2_rel)
    p = _seg_sum(h, src, dst, zer)
    h = _comb(p, r, W2_rel, g2, be2, Wlin, blin, False)
    r = _root(h, W3_root, b3_rel)
    p = _seg_sum(h, src, dst, zer)
    out = _comb(p, r, W3_rel, g3, be3, Wlin, blin, True)
    return out


# speedup vs baseline: 1.0237x; 1.0237x over previous
import functools
import jax
import jax.numpy as jnp
from jax import lax
from jax.experimental import pallas as pl
from jax.experimental.pallas import tpu as pltpu
from jax.experimental.pallas import tpu_sc as plsc

_N = 10000
_E = 320000
_D = 128
_NC = 2
_NS = 16
_NW = _NC * _NS
_CH = 200
_NCHUNK = 50
_EPT = _NCHUNK * _CH
_EPAD = _NW * _EPT
_NPAD = 10240
_RPT = _NPAD // _NS


def _seg_sum_body(y_hbm, src_hbm, dst_hbm, zer_hbm, out_hbm,
                  acc, srcb0, dstb0, srcb1, dstb1, gbuf, gsem, isem0, isem1):
    c = lax.axis_index("c")
    s = lax.axis_index("s")
    cbase = (c * _NS + s) * _EPT

    def istart(g, sb, db, isem):
        off = cbase + g * _CH
        pltpu.async_copy(src_hbm.at[pl.ds(off, _CH)], sb, isem)
        pltpu.async_copy(dst_hbm.at[pl.ds(off, _CH)], db, isem)

    def iwait(g, sb, db, isem):
        off = cbase + g * _CH
        pltpu.make_async_copy(src_hbm.at[pl.ds(off, _CH)], sb, isem).wait()
        pltpu.make_async_copy(dst_hbm.at[pl.ds(off, _CH)], db, isem).wait()

    istart(0, srcb0, dstb0, isem0)
    istart(1, srcb1, dstb1, isem1)
    pltpu.sync_copy(zer_hbm, acc.at[pl.ds(s * _RPT, _RPT)])
    plsc.subcore_barrier()

    def chunk(g, sb, db, isem, k):
        iwait(g, sb, db, isem)
        pltpu.async_copy(y_hbm.at[sb], gbuf, gsem).wait()
        pltpu.sync_copy(gbuf, acc.at[db], add=True)

        @pl.when(k < _NCHUNK // 2 - 1)
        def _():
            istart(g + 2, sb, db, isem)

    def body(k, carry):
        chunk(2 * k, srcb0, dstb0, isem0, k)
        chunk(2 * k + 1, srcb1, dstb1, isem1, k)
        return carry

    lax.fori_loop(0, _NCHUNK // 2, body, 0)
    plsc.subcore_barrier()
    pltpu.sync_copy(acc.at[pl.ds(s * _RPT, _RPT)],
                    out_hbm.at[c, pl.ds(s * _RPT, _RPT)])


_seg_sum = functools.partial(
    pl.kernel,
    out_type=jax.ShapeDtypeStruct((_NC, _NPAD, _D), jnp.float32),
    mesh=plsc.VectorSubcoreMesh(core_axis_name="c", subcore_axis_name="s"),
    scratch_types=[
        pltpu.VMEM_SHARED((_NPAD, _D), jnp.float32),
        pltpu.VMEM((_CH,), jnp.int32),
        pltpu.VMEM((_CH,), jnp.int32),
        pltpu.VMEM((_CH,), jnp.int32),
        pltpu.VMEM((_CH,), jnp.int32),
        pltpu.VMEM((_CH, _D), jnp.float32),
        pltpu.SemaphoreType.DMA,
        pltpu.SemaphoreType.DMA,
        pltpu.SemaphoreType.DMA,
    ],
)(_seg_sum_body)


_BLK = 2000


def _dense_body(final, p_ref, x_ref, wr_ref, br_ref, wt_ref, g_ref, be_ref,
                wl_ref, bl_ref, o_ref):
    agg = p_ref[0] + p_ref[1]
    h = (jnp.dot(agg, wr_ref[...], preferred_element_type=jnp.float32)
         + jnp.dot(x_ref[...], wt_ref[...], preferred_element_type=jnp.float32)
         + br_ref[...])
    m = jnp.mean(h, axis=-1, keepdims=True)
    v = jnp.mean((h - m) * (h - m), axis=-1, keepdims=True)
    ln = (h - m) * lax.rsqrt(v + 1e-5) * g_ref[...] + be_ref[...]
    if final:
        o_ref[...] = (jnp.dot(ln, wl_ref[...],
                              preferred_element_type=jnp.float32)
                      + bl_ref[...])
    else:
        o_ref[...] = ln


def _dense(p, x, w_rel, b_rel, w_root, g, be, w_lin, b_lin, final):
    vec = pl.BlockSpec((1, _D), lambda i: (0, 0))
    mat = pl.BlockSpec((_D, _D), lambda i: (0, 0))
    return pl.pallas_call(
        functools.partial(_dense_body, final),
        grid=(_N // _BLK,),
        in_specs=[
            pl.BlockSpec((2, _BLK, _D), lambda i: (0, i, 0)),
            pl.BlockSpec((_BLK, _D), lambda i: (i, 0)),
            mat, vec, mat, vec, vec, mat, vec,
        ],
        out_specs=pl.BlockSpec((_BLK, _D), lambda i: (i, 0)),
        out_shape=jax.ShapeDtypeStruct((_N, _D), jnp.float32),
    )(p, x, w_rel, b_rel.reshape(1, _D), w_root, g.reshape(1, _D),
      be.reshape(1, _D), w_lin, b_lin.reshape(1, _D))


def kernel(x, edge_index, batch,
           W1_rel, b1_rel, W1_root, g1, be1,
           W2_rel, b2_rel, W2_root, g2, be2,
           W3_rel, b3_rel, W3_root, g3, be3,
           Wlin, blin):
    del batch
    src = edge_index[0]
    dst = edge_index[1]
    zer = jnp.zeros((_RPT, _D), jnp.float32)

    p = _seg_sum(x, src, dst, zer)
    h = _dense(p, x, W1_rel, b1_rel, W1_root, g1, be1, Wlin, blin, False)
    p = _seg_sum(h, src, dst, zer)
    h = _dense(p, h, W2_rel, b2_rel, W2_root, g2, be2, Wlin, blin, False)
    p = _seg_sum(h, src, dst, zer)
    out = _dense(p, h, W3_rel, b3_rel, W3_root, g3, be3, Wlin, blin, True)
    return out


# speedup vs baseline: 1.0331x; 1.0092x over previous
import functools
import jax
import jax.numpy as jnp
from jax import lax
from jax.experimental import pallas as pl
from jax.experimental.pallas import tpu as pltpu
from jax.experimental.pallas import tpu_sc as plsc

_N = 10000
_E = 320000
_D = 128
_NC = 2
_NS = 16
_NW = _NC * _NS
_CH = 200
_NCHUNK = 50
_EPT = _NCHUNK * _CH
_EPAD = _NW * _EPT
_NPAD = 10240
_RPT = _NPAD // _NS


def _seg_sum_body(y_hbm, src_hbm, dst_hbm, zer_hbm, out_hbm,
                  acc, srcb0, dstb0, srcb1, dstb1, gbuf, gsem, isem0, isem1):
    c = lax.axis_index("c")
    s = lax.axis_index("s")
    cbase = (c * _NS + s) * _EPT

    def istart(g, sb, db, isem):
        off = cbase + g * _CH
        pltpu.async_copy(src_hbm.at[pl.ds(off, _CH)], sb, isem)
        pltpu.async_copy(dst_hbm.at[pl.ds(off, _CH)], db, isem)

    def iwait(g, sb, db, isem):
        off = cbase + g * _CH
        pltpu.make_async_copy(src_hbm.at[pl.ds(off, _CH)], sb, isem).wait()
        pltpu.make_async_copy(dst_hbm.at[pl.ds(off, _CH)], db, isem).wait()

    istart(0, srcb0, dstb0, isem0)
    istart(1, srcb1, dstb1, isem1)
    pltpu.sync_copy(zer_hbm, acc.at[pl.ds(s * _RPT, _RPT)])
    plsc.subcore_barrier()

    def chunk(g, sb, db, isem, k):
        iwait(g, sb, db, isem)
        pltpu.async_copy(y_hbm.at[sb], gbuf, gsem).wait()
        pltpu.sync_copy(gbuf, acc.at[db], add=True)

        @pl.when(k < _NCHUNK // 2 - 1)
        def _():
            istart(g + 2, sb, db, isem)

    def body(k, carry):
        chunk(2 * k, srcb0, dstb0, isem0, k)
        chunk(2 * k + 1, srcb1, dstb1, isem1, k)
        return carry

    lax.fori_loop(0, _NCHUNK // 2, body, 0)
    plsc.subcore_barrier()
    pltpu.sync_copy(acc.at[pl.ds(s * _RPT, _RPT)],
                    out_hbm.at[c, pl.ds(s * _RPT, _RPT)])


_seg_sum = functools.partial(
    pl.kernel,
    out_type=jax.ShapeDtypeStruct((_NC, _NPAD, _D), jnp.float32),
    mesh=plsc.VectorSubcoreMesh(core_axis_name="c", subcore_axis_name="s"),
    scratch_types=[
        pltpu.VMEM_SHARED((_NPAD, _D), jnp.float32),
        pltpu.VMEM((_CH,), jnp.int32),
        pltpu.VMEM((_CH,), jnp.int32),
        pltpu.VMEM((_CH,), jnp.int32),
        pltpu.VMEM((_CH,), jnp.int32),
        pltpu.VMEM((_CH, _D), jnp.float32),
        pltpu.SemaphoreType.DMA,
        pltpu.SemaphoreType.DMA,
        pltpu.SemaphoreType.DMA,
    ],
)(_seg_sum_body)


_BLK = 5000


def _dense_body(final, p_ref, x_ref, wr_ref, br_ref, wt_ref, g_ref, be_ref,
                wl_ref, bl_ref, o_ref):
    agg = p_ref[0] + p_ref[1]
    h = (jnp.dot(agg, wr_ref[...], preferred_element_type=jnp.float32)
         + jnp.dot(x_ref[...], wt_ref[...], preferred_element_type=jnp.float32)
         + br_ref[...])
    m = jnp.mean(h, axis=-1, keepdims=True)
    v = jnp.mean((h - m) * (h - m), axis=-1, keepdims=True)
    ln = (h - m) * lax.rsqrt(v + 1e-5) * g_ref[...] + be_ref[...]
    if final:
        o_ref[...] = (jnp.dot(ln, wl_ref[...],
                              preferred_element_type=jnp.float32)
                      + bl_ref[...])
    else:
        o_ref[...] = ln


def _dense(p, x, w_rel, b_rel, w_root, g, be, w_lin, b_lin, final):
    vec = pl.BlockSpec((1, _D), lambda i: (0, 0))
    mat = pl.BlockSpec((_D, _D), lambda i: (0, 0))
    return pl.pallas_call(
        functools.partial(_dense_body, final),
        grid=(_N // _BLK,),
        in_specs=[
            pl.BlockSpec((2, _BLK, _D), lambda i: (0, i, 0)),
            pl.BlockSpec((_BLK, _D), lambda i: (i, 0)),
            mat, vec, mat, vec, vec, mat, vec,
        ],
        out_specs=pl.BlockSpec((_BLK, _D), lambda i: (i, 0)),
        out_shape=jax.ShapeDtypeStruct((_N, _D), jnp.float32),
    )(p, x, w_rel, b_rel.reshape(1, _D), w_root, g.reshape(1, _D),
      be.reshape(1, _D), w_lin, b_lin.reshape(1, _D))


def kernel(x, edge_index, batch,
           W1_rel, b1_rel, W1_root, g1, be1,
           W2_rel, b2_rel, W2_root, g2, be2,
           W3_rel, b3_rel, W3_root, g3, be3,
           Wlin, blin):
    del batch
    src = edge_index[0]
    dst = edge_index[1]
    zer = jnp.zeros((_RPT, _D), jnp.float32)

    p = _seg_sum(x, src, dst, zer)
    h = _dense(p, x, W1_rel, b1_rel, W1_root, g1, be1, Wlin, blin, False)
    p = _seg_sum(h, src, dst, zer)
    h = _dense(p, h, W2_rel, b2_rel, W2_root, g2, be2, Wlin, blin, False)
    p = _seg_sum(h, src, dst, zer)
    out = _dense(p, h, W3_rel, b3_rel, W3_root, g3, be3, Wlin, blin, True)
    return out
